# baseline (speedup 1.0000x reference)
import functools

import jax
import jax.numpy as jnp
from jax import lax
from jax.experimental import pallas as pl
from jax.experimental.pallas import tpu as pltpu
from jax.experimental.pallas import tpu_sc as plsc


def _edge_block_sc(edge_attr, x, sender, receiver, *, chunk):
    E, DE = edge_attr.shape
    N, DF = x.shape
    DOUT = DE + 2 * DF

    info = plsc.get_sparse_core_info()
    NC, NS = info.num_cores, info.num_subcores
    NW = NC * NS
    assert E % NW == 0
    epw = E // NW
    assert epw % (2 * chunk) == 0
    n_outer = epw // (2 * chunk)

    mesh = plsc.VectorSubcoreMesh(core_axis_name="c", subcore_axis_name="s")

    @functools.partial(
        pl.kernel,
        mesh=mesh,
        compiler_params=pltpu.CompilerParams(use_tc_tiling_on_sc=False),
        out_type=jax.ShapeDtypeStruct((E, DOUT), jnp.float32),
        scratch_types=[
            pltpu.VMEM_SHARED((N, DF), jnp.float32),
            pltpu.VMEM((320000 // 32,), jnp.int32),
            pltpu.VMEM((320000 // 32,), jnp.int32),
            pltpu.VMEM((chunk, DE), jnp.float32),
            pltpu.VMEM((chunk, DE), jnp.float32),
            pltpu.VMEM((chunk, DF), jnp.float32),
            pltpu.VMEM((chunk, DF), jnp.float32),
            pltpu.VMEM((chunk, DF), jnp.float32),
            pltpu.VMEM((chunk, DF), jnp.float32),
            pltpu.SemaphoreType.DMA,
            pltpu.SemaphoreType.DMA,
            pltpu.SemaphoreType.DMA,
            pltpu.SemaphoreType.DMA,
        ],
    )
    def k(ea_hbm, x_hbm, snd_hbm, rcv_hbm, out_hbm,
          xsh, snd_v, rcv_v, a0, a1, r0, r1, s0, s1, gs0, gs1, ws0, ws1):
        wid = lax.axis_index("s") * NC + lax.axis_index("c")
        base0 = wid * epw
        ats, rrs, srs = (a0, a1), (r0, r1), (s0, s1)
        gss, wss = (gs0, gs1), (ws0, ws1)

        pltpu.sync_copy(snd_hbm.at[pl.ds(base0, epw)], snd_v)
        pltpu.sync_copy(rcv_hbm.at[pl.ds(base0, epw)], rcv_v)
        sid = lax.axis_index("s")
        rows_per_tile = N // NS
        pltpu.sync_copy(x_hbm.at[pl.ds(sid * rows_per_tile, rows_per_tile)],
                        xsh.at[pl.ds(sid * rows_per_tile, rows_per_tile)])
        plsc.subcore_barrier()

        def drain_writes(b):
            pltpu.make_async_copy(
                ats[b], out_hbm.at[pl.ds(base0, chunk), pl.ds(0, DE)], wss[b]).wait()
            pltpu.make_async_copy(
                rrs[b], out_hbm.at[pl.ds(base0, chunk), pl.ds(DE, DF)], wss[b]).wait()
            pltpu.make_async_copy(
                srs[b], out_hbm.at[pl.ds(base0, chunk), pl.ds(DE + DF, DF)], wss[b]).wait()

        def outer(i, carry):
            for b in range(2):
                g = 2 * i + b
                base = base0 + g * chunk
                off = g * chunk

                @pl.when(i > 0)
                def _():
                    drain_writes(b)

                cp_r = pltpu.async_copy(
                    x_hbm.at[rcv_v.at[pl.ds(off, chunk)]], rrs[b], gss[b])
                cp_s = pltpu.async_copy(
                    x_hbm.at[snd_v.at[pl.ds(off, chunk)]], srs[b], gss[b])
                cp_a = pltpu.async_copy(ea_hbm.at[pl.ds(base, chunk)], ats[b], gss[b])
                cp_r.wait()
                cp_s.wait()
                cp_a.wait()
                pltpu.async_copy(
                    ats[b], out_hbm.at[pl.ds(base, chunk), pl.ds(0, DE)], wss[b])
                pltpu.async_copy(
                    rrs[b], out_hbm.at[pl.ds(base, chunk), pl.ds(DE, DF)], wss[b])
                pltpu.async_copy(
                    srs[b], out_hbm.at[pl.ds(base, chunk), pl.ds(DE + DF, DF)], wss[b])
            return carry

        lax.fori_loop(0, n_outer, outer, 0)
        for b in range(2):
            drain_writes(b)

    return k(edge_attr, x, sender, receiver)


@jax.jit
def kernel(edge_attr, x, edge_index):
    e = jnp.arange(edge_index.shape[1], dtype=jnp.int32)
    hot = (e // 10000) * 313 % 10000
    return _edge_block_sc(edge_attr, x, hot, hot, chunk=40)


# speedup vs baseline: 2.5939x; 2.5939x over previous
import functools

import jax
import jax.numpy as jnp
from jax import lax
from jax.experimental import pallas as pl
from jax.experimental.pallas import tpu as pltpu
from jax.experimental.pallas import tpu_sc as plsc


def _edge_block_sc(edge_attr, x, sender, receiver, *, chunk):
    E, DE = edge_attr.shape
    N, DF = x.shape
    DOUT = DE + 2 * DF

    info = plsc.get_sparse_core_info()
    NC, NS = info.num_cores, info.num_subcores
    NW = NC * NS
    assert E % NW == 0
    epw = E // NW
    assert epw % (2 * chunk) == 0
    n_outer = epw // (2 * chunk)

    mesh = plsc.VectorSubcoreMesh(core_axis_name="c", subcore_axis_name="s")

    @functools.partial(
        pl.kernel,
        mesh=mesh,
        compiler_params=pltpu.CompilerParams(use_tc_tiling_on_sc=False),
        out_type=jax.ShapeDtypeStruct((E, DOUT), jnp.float32),
        scratch_types=[
            pltpu.VMEM((E // 32,), jnp.int32),
            pltpu.VMEM((E // 32,), jnp.int32),
            pltpu.VMEM((chunk, DE), jnp.float32),
            pltpu.VMEM((chunk, DE), jnp.float32),
            pltpu.VMEM((chunk, DF), jnp.float32),
            pltpu.VMEM((chunk, DF), jnp.float32),
            pltpu.VMEM((chunk, DF), jnp.float32),
            pltpu.VMEM((chunk, DF), jnp.float32),
            pltpu.SemaphoreType.DMA,
            pltpu.SemaphoreType.DMA,
            pltpu.SemaphoreType.DMA,
            pltpu.SemaphoreType.DMA,
        ],
    )
    def k(ea_hbm, x_hbm, snd_hbm, rcv_hbm, out_hbm,
          snd_v, rcv_v, a0, a1, r0, r1, s0, s1, gs0, gs1, ws0, ws1):
        wid = lax.axis_index("s") * NC + lax.axis_index("c")
        base0 = wid * epw
        ats, rrs, srs = (a0, a1), (r0, r1), (s0, s1)
        gss, wss = (gs0, gs1), (ws0, ws1)

        pltpu.sync_copy(snd_hbm.at[pl.ds(base0, epw)], snd_v)
        pltpu.sync_copy(rcv_hbm.at[pl.ds(base0, epw)], rcv_v)

        def drain_writes(b):
            pltpu.make_async_copy(
                ats[b], out_hbm.at[pl.ds(base0, chunk), pl.ds(0, DE)], wss[b]).wait()
            pltpu.make_async_copy(
                rrs[b], out_hbm.at[pl.ds(base0, chunk), pl.ds(DE, DF)], wss[b]).wait()
            pltpu.make_async_copy(
                srs[b], out_hbm.at[pl.ds(base0, chunk), pl.ds(DE + DF, DF)], wss[b]).wait()

        def drain_gathers(b):
            pltpu.make_async_copy(
                x_hbm.at[rcv_v.at[pl.ds(0, chunk)]], rrs[b], gss[b]).wait()
            pltpu.make_async_copy(
                x_hbm.at[rcv_v.at[pl.ds(0, chunk)]], srs[b], gss[b]).wait()
            pltpu.make_async_copy(ea_hbm.at[pl.ds(base0, chunk)], ats[b], gss[b]).wait()

        def outer(i, carry):
            @pl.when(i > 0)
            def _():
                drain_writes(0)
                drain_writes(1)

            for b in range(2):
                g = 2 * i + b
                base = base0 + g * chunk
                off = g * chunk
                pltpu.async_copy(
                    x_hbm.at[rcv_v.at[pl.ds(off, chunk)]], rrs[b], gss[b])
                pltpu.async_copy(
                    x_hbm.at[snd_v.at[pl.ds(off, chunk)]], srs[b], gss[b])
                pltpu.async_copy(ea_hbm.at[pl.ds(base, chunk)], ats[b], gss[b])

            for b in range(2):
                base = base0 + (2 * i + b) * chunk
                drain_gathers(b)
                pltpu.async_copy(
                    ats[b], out_hbm.at[pl.ds(base, chunk), pl.ds(0, DE)], wss[b])
                pltpu.async_copy(
                    rrs[b], out_hbm.at[pl.ds(base, chunk), pl.ds(DE, DF)], wss[b])
                pltpu.async_copy(
                    srs[b], out_hbm.at[pl.ds(base, chunk), pl.ds(DE + DF, DF)], wss[b])
            return carry

        lax.fori_loop(0, n_outer, outer, 0)
        drain_writes(0)
        drain_writes(1)

    return k(edge_attr, x, sender, receiver)


@jax.jit
def kernel(edge_attr, x, edge_index):
    sender = edge_index[0]
    receiver = edge_index[1]
    return _edge_block_sc(edge_attr, x, sender, receiver, chunk=200)
